# baseline (speedup 1.0000x reference)
import functools

import jax
import jax.numpy as jnp
from jax import lax
from jax.experimental import pallas as pl
from jax.experimental.pallas import tpu as pltpu
from jax.experimental.pallas import tpu_sc as plsc

T, D, I, E, K = 2048, 1024, 512, 16, 2
BM = 256
NBLK = (K * T) // BM + E
NPAD = NBLK * BM
NEG_INF = -1e30
NW = 32
APW = (K * T) // NW
CHS = 64


def _router_sort_kernel(x_ref, w_ref, b_ref, slots_ref, wts_ref,
                        counts_ref, ent_ref, be_ref, nb_ref):
    x = x_ref[...]
    logits = jnp.dot(x, w_ref[...], preferred_element_type=jnp.float32)
    biased = logits + b_ref[...]

    iota_e = lax.broadcasted_iota(jnp.int32, (T, E), 1)
    m0 = jnp.max(biased, axis=-1, keepdims=True)
    e0 = jnp.min(jnp.where(biased == m0, iota_e, E), axis=-1, keepdims=True)
    masked = jnp.where(iota_e == e0, NEG_INF, biased)
    m1 = jnp.max(masked, axis=-1, keepdims=True)
    e1 = jnp.min(jnp.where(masked == m1, iota_e, E), axis=-1, keepdims=True)

    one0 = (iota_e == e0).astype(jnp.float32)
    one1 = (iota_e == e1).astype(jnp.float32)
    u0 = jnp.sum(jnp.where(iota_e == e0, logits, 0.0), axis=-1, keepdims=True)
    u1 = jnp.sum(jnp.where(iota_e == e1, logits, 0.0), axis=-1, keepdims=True)
    s0 = jax.nn.sigmoid(u0)
    s1 = jax.nn.sigmoid(u1)
    denom = s0 + s1
    wts_ref[...] = jnp.concatenate([s0 / denom, s1 / denom], axis=1)

    A = one0 + one1
    counts = jnp.sum(A, axis=0, keepdims=True)
    counts_ref[...] = counts
    total = jnp.maximum(jnp.sum(counts), 1.0)
    frac = counts / total
    ent_ref[...] = -jnp.sum(frac * jnp.log(frac + 1e-6), keepdims=True).reshape(1, 1)

    CB = 512
    ib = lax.broadcasted_iota(jnp.int32, (CB, CB), 0)
    jb = lax.broadcasted_iota(jnp.int32, (CB, CB), 1)
    Lb = (jb < ib).astype(jnp.float32)
    carry = jnp.zeros((1, E), jnp.float32)
    chunks = []
    for c in range(T // CB):
        Ac = A[c * CB:(c + 1) * CB]
        chunks.append(jnp.dot(Lb, Ac, preferred_element_type=jnp.float32) + carry)
        carry = carry + jnp.sum(Ac, axis=0, keepdims=True)
    P = jnp.concatenate(chunks, axis=0)

    counts_pad = jnp.ceil(counts / BM) * BM
    iu = lax.broadcasted_iota(jnp.int32, (E, E), 0)
    ju = lax.broadcasted_iota(jnp.int32, (E, E), 1)
    U = (iu < ju).astype(jnp.float32)
    off = jnp.dot(counts_pad, U, preferred_element_type=jnp.float32)

    base = off + P
    slot0 = jnp.sum(jnp.where(iota_e == e0, base, 0.0), axis=-1, keepdims=True)
    slot1 = jnp.sum(jnp.where(iota_e == e1, base, 0.0), axis=-1, keepdims=True)
    slots_ref[...] = jnp.concatenate([slot0, slot1], axis=1).astype(jnp.int32)

    bstart = off / BM
    iblk = lax.broadcasted_iota(jnp.int32, (NBLK, E), 0).astype(jnp.float32)
    be = jnp.sum((iblk >= bstart).astype(jnp.int32), axis=-1, keepdims=True) - 1
    be_ref[...] = be
    nblocks = (jnp.sum(counts_pad) / BM).astype(jnp.int32)
    nb_ref[...] = nblocks.reshape(1, 1)


def _sc_scatter_x(x_hbm, slots_hbm, tok_hbm, xg_hbm, tok_v, slot_v, rows_v, sem):
    wid = lax.axis_index("s") * 2 + lax.axis_index("c")
    for r in range(APW // CHS):
        a_base = wid * APW + r * CHS
        pltpu.sync_copy(slots_hbm.at[pl.ds(a_base, CHS)], slot_v)
        pltpu.sync_copy(tok_hbm.at[pl.ds(a_base, CHS)], tok_v)
        pltpu.async_copy(x_hbm.at[tok_v], rows_v, sem).wait()
        pltpu.async_copy(rows_v, xg_hbm.at[slot_v], sem).wait()


def _sc_gather_y(y_hbm, slots_hbm, out_hbm, idx_v, rows_v, sem):
    wid = lax.axis_index("s") * 2 + lax.axis_index("c")
    for r in range(APW // CHS):
        a_base = wid * APW + r * CHS
        pltpu.sync_copy(slots_hbm.at[pl.ds(a_base, CHS)], idx_v)
        pltpu.async_copy(y_hbm.at[idx_v], rows_v, sem).wait()
        pltpu.sync_copy(rows_v, out_hbm.at[pl.ds(a_base, CHS)])


def _combine_kernel(yg_ref, wts_ref, out_ref):
    y0 = yg_ref[:, 0, :]
    y1 = yg_ref[:, 1, :]
    w0 = wts_ref[:, 0:1]
    w1 = wts_ref[:, 1:2]
    out_ref[...] = w0 * y0 + w1 * y1


def _gmm_kernel(be_ref, nb_ref, xg_ref, wgu_ref, wd_ref, y_ref):
    @pl.when(pl.program_id(0) < nb_ref[0])
    def _():
        xb = xg_ref[...].astype(jnp.bfloat16)
        wgu = wgu_ref[0].astype(jnp.bfloat16)
        gu = jnp.dot(xb, wgu, preferred_element_type=jnp.float32)
        gate = gu[:, :I]
        up = gu[:, I:]
        h = (gate * jax.nn.sigmoid(gate) * up).astype(jnp.bfloat16)
        wd = wd_ref[0].astype(jnp.bfloat16)
        y_ref[...] = jnp.dot(h, wd, preferred_element_type=jnp.float32)


_SC_MESH = plsc.VectorSubcoreMesh(core_axis_name="c", subcore_axis_name="s")

_scatter_x = functools.partial(
    pl.kernel,
    out_type=jax.ShapeDtypeStruct((NPAD, D), jnp.float32),
    mesh=_SC_MESH,
    scratch_types=[
        pltpu.VMEM((CHS,), jnp.int32),
        pltpu.VMEM((CHS,), jnp.int32),
        pltpu.VMEM((CHS, D), jnp.float32),
        pltpu.SemaphoreType.DMA,
    ],
)(_sc_scatter_x)

_gather_y = functools.partial(
    pl.kernel,
    out_type=jax.ShapeDtypeStruct((K * T, D), jnp.float32),
    mesh=_SC_MESH,
    scratch_types=[
        pltpu.VMEM((CHS,), jnp.int32),
        pltpu.VMEM((CHS, D), jnp.float32),
        pltpu.SemaphoreType.DMA,
    ],
)(_sc_gather_y)


@jax.jit
def kernel(x, router, router_bias, w_gate_up, w_down):
    x_flat = x.reshape(T, D)

    slots2, wts2, counts, ent, be, nb = pl.pallas_call(
        _router_sort_kernel,
        out_shape=[
            jax.ShapeDtypeStruct((T, K), jnp.int32),
            jax.ShapeDtypeStruct((T, K), jnp.float32),
            jax.ShapeDtypeStruct((1, E), jnp.float32),
            jax.ShapeDtypeStruct((1, 1), jnp.float32),
            jax.ShapeDtypeStruct((NBLK, 1), jnp.int32),
            jax.ShapeDtypeStruct((1, 1), jnp.int32),
        ],
    )(x_flat, router, router_bias.reshape(1, E))

    slots_flat = slots2.reshape(K * T)
    be_flat = be.reshape(NBLK)
    nb_flat = nb.reshape(1)

    tok_idx = jnp.arange(K * T, dtype=jnp.int32) // K
    xg = _scatter_x(x_flat, slots_flat, tok_idx)

    y = pl.pallas_call(
        _gmm_kernel,
        grid_spec=pltpu.PrefetchScalarGridSpec(
            num_scalar_prefetch=2,
            grid=(NBLK,),
            in_specs=[
                pl.BlockSpec((BM, D), lambda i, be, nb: (i, 0)),
                pl.BlockSpec((1, D, 2 * I), lambda i, be, nb: (be[i], 0, 0)),
                pl.BlockSpec((1, I, D), lambda i, be, nb: (be[i], 0, 0)),
            ],
            out_specs=pl.BlockSpec((BM, D), lambda i, be, nb: (i, 0)),
        ),
        out_shape=jax.ShapeDtypeStruct((NPAD, D), jnp.float32),
    )(be_flat, nb_flat, xg, w_gate_up, w_down)

    y_gath = _gather_y(y, slots_flat)

    routed = pl.pallas_call(
        _combine_kernel,
        grid=(4,),
        in_specs=[
            pl.BlockSpec((T // 4, K, D), lambda t: (t, 0, 0)),
            pl.BlockSpec((T // 4, K), lambda t: (t, 0)),
        ],
        out_specs=pl.BlockSpec((T // 4, D), lambda t: (t, 0)),
        out_shape=jax.ShapeDtypeStruct((T, D), jnp.float32),
    )(y_gath.reshape(T, K, D), wts2)

    return routed.reshape(x.shape), counts.reshape(E), ent.reshape(())


# speedup vs baseline: 1.0494x; 1.0494x over previous
import functools

import jax
import jax.numpy as jnp
from jax import lax
from jax.experimental import pallas as pl
from jax.experimental.pallas import tpu as pltpu
from jax.experimental.pallas import tpu_sc as plsc

T, D, I, E, K = 2048, 1024, 512, 16, 2
BM = 256
NBLK = (K * T) // BM + E
NPAD = NBLK * BM
NEG_INF = -1e30
NW = 32
APW = (K * T) // NW
CHS = 64


def _router_sort_kernel(x_ref, w_ref, b_ref, slots_ref, wts_ref,
                        counts_ref, ent_ref, be_ref, nb_ref):
    x = x_ref[...]
    logits = jnp.dot(x, w_ref[...], preferred_element_type=jnp.float32)
    biased = logits + b_ref[...]

    iota_e = lax.broadcasted_iota(jnp.int32, (T, E), 1)
    m0 = jnp.max(biased, axis=-1, keepdims=True)
    e0 = jnp.min(jnp.where(biased == m0, iota_e, E), axis=-1, keepdims=True)
    masked = jnp.where(iota_e == e0, NEG_INF, biased)
    m1 = jnp.max(masked, axis=-1, keepdims=True)
    e1 = jnp.min(jnp.where(masked == m1, iota_e, E), axis=-1, keepdims=True)

    one0 = (iota_e == e0).astype(jnp.float32)
    one1 = (iota_e == e1).astype(jnp.float32)
    u0 = jnp.sum(jnp.where(iota_e == e0, logits, 0.0), axis=-1, keepdims=True)
    u1 = jnp.sum(jnp.where(iota_e == e1, logits, 0.0), axis=-1, keepdims=True)
    s0 = jax.nn.sigmoid(u0)
    s1 = jax.nn.sigmoid(u1)
    denom = s0 + s1
    wts_ref[...] = jnp.concatenate([s0 / denom, s1 / denom], axis=1)

    A = one0 + one1
    counts = jnp.sum(A, axis=0, keepdims=True)
    counts_ref[...] = counts
    total = jnp.maximum(jnp.sum(counts), 1.0)
    frac = counts / total
    ent_ref[...] = -jnp.sum(frac * jnp.log(frac + 1e-6), keepdims=True).reshape(1, 1)

    CB = 512
    ib = lax.broadcasted_iota(jnp.int32, (CB, CB), 0)
    jb = lax.broadcasted_iota(jnp.int32, (CB, CB), 1)
    Lb = (jb < ib).astype(jnp.float32)
    carry = jnp.zeros((1, E), jnp.float32)
    chunks = []
    for c in range(T // CB):
        Ac = A[c * CB:(c + 1) * CB]
        chunks.append(jnp.dot(Lb, Ac, preferred_element_type=jnp.float32) + carry)
        carry = carry + jnp.sum(Ac, axis=0, keepdims=True)
    P = jnp.concatenate(chunks, axis=0)

    counts_pad = jnp.ceil(counts / BM) * BM
    iu = lax.broadcasted_iota(jnp.int32, (E, E), 0)
    ju = lax.broadcasted_iota(jnp.int32, (E, E), 1)
    U = (iu < ju).astype(jnp.float32)
    off = jnp.dot(counts_pad, U, preferred_element_type=jnp.float32)

    base = off + P
    slot0 = jnp.sum(jnp.where(iota_e == e0, base, 0.0), axis=-1, keepdims=True)
    slot1 = jnp.sum(jnp.where(iota_e == e1, base, 0.0), axis=-1, keepdims=True)
    slots_ref[...] = jnp.concatenate([slot0, slot1], axis=1).astype(jnp.int32)

    bstart = off / BM
    iblk = lax.broadcasted_iota(jnp.int32, (NBLK, E), 0).astype(jnp.float32)
    be = jnp.sum((iblk >= bstart).astype(jnp.int32), axis=-1, keepdims=True) - 1
    be_ref[...] = be
    nblocks = (jnp.sum(counts_pad) / BM).astype(jnp.int32)
    nb_ref[...] = nblocks.reshape(1, 1)


def _sc_scatter_x(x_hbm, slots_hbm, tok_hbm, xg_hbm, tok_v, slot_v, rows_v, sem):
    wid = lax.axis_index("s") * 2 + lax.axis_index("c")
    for r in range(APW // CHS):
        a_base = wid * APW + r * CHS
        pltpu.sync_copy(slots_hbm.at[pl.ds(a_base, CHS)], slot_v)
        pltpu.sync_copy(tok_hbm.at[pl.ds(a_base, CHS)], tok_v)
        pltpu.async_copy(x_hbm.at[tok_v], rows_v, sem).wait()
        pltpu.async_copy(rows_v, xg_hbm.at[slot_v], sem).wait()


def _sc_gather_y(y_hbm, slots_hbm, out_hbm, idx_v, rows_v, sem):
    wid = lax.axis_index("s") * 2 + lax.axis_index("c")
    for r in range(APW // CHS):
        a_base = wid * APW + r * CHS
        pltpu.sync_copy(slots_hbm.at[pl.ds(a_base, CHS)], idx_v)
        pltpu.async_copy(y_hbm.at[idx_v], rows_v, sem).wait()
        pltpu.sync_copy(rows_v, out_hbm.at[pl.ds(a_base, CHS)])


def _combine_kernel(yg_ref, wts_ref, out_ref):
    y0 = yg_ref[:, 0, :]
    y1 = yg_ref[:, 1, :]
    w0 = wts_ref[:, 0:1]
    w1 = wts_ref[:, 1:2]
    out_ref[...] = w0 * y0 + w1 * y1


def _gmm_kernel(be_ref, nb_ref, xg_ref, wgu_ref, wd_ref, y_ref):
    @pl.when(pl.program_id(0) < nb_ref[0])
    def _():
        xb = xg_ref[...].astype(jnp.bfloat16)
        wgu = wgu_ref[0].astype(jnp.bfloat16)
        gu = jnp.dot(xb, wgu, preferred_element_type=jnp.float32)
        gate = gu[:, :I]
        up = gu[:, I:]
        h = (gate * jax.nn.sigmoid(gate) * up).astype(jnp.bfloat16)
        wd = wd_ref[0].astype(jnp.bfloat16)
        y_ref[...] = jnp.dot(h, wd, preferred_element_type=jnp.float32)


_SC_MESH = plsc.VectorSubcoreMesh(core_axis_name="c", subcore_axis_name="s")

_scatter_x = functools.partial(
    pl.kernel,
    out_type=jax.ShapeDtypeStruct((NPAD, D), jnp.float32),
    mesh=_SC_MESH,
    scratch_types=[
        pltpu.VMEM((CHS,), jnp.int32),
        pltpu.VMEM((CHS,), jnp.int32),
        pltpu.VMEM((CHS, D), jnp.float32),
        pltpu.SemaphoreType.DMA,
    ],
)(_sc_scatter_x)

_gather_y = functools.partial(
    pl.kernel,
    out_type=jax.ShapeDtypeStruct((K * T, D), jnp.float32),
    mesh=_SC_MESH,
    scratch_types=[
        pltpu.VMEM((CHS,), jnp.int32),
        pltpu.VMEM((CHS, D), jnp.float32),
        pltpu.SemaphoreType.DMA,
    ],
)(_sc_gather_y)


@jax.jit
def kernel(x, router, router_bias, w_gate_up, w_down):
    x_flat = x.reshape(T, D)

    slots2, wts2, counts, ent, be, nb = pl.pallas_call(
        _router_sort_kernel,
        out_shape=[
            jax.ShapeDtypeStruct((T, K), jnp.int32),
            jax.ShapeDtypeStruct((T, K), jnp.float32),
            jax.ShapeDtypeStruct((1, E), jnp.float32),
            jax.ShapeDtypeStruct((1, 1), jnp.float32),
            jax.ShapeDtypeStruct((NBLK, 1), jnp.int32),
            jax.ShapeDtypeStruct((1, 1), jnp.int32),
        ],
    )(x_flat, router, router_bias.reshape(1, E))

    slots_flat = slots2.reshape(K * T)
    be_flat = be.reshape(NBLK)
    nb_flat = nb.reshape(1)

    tok_idx = jnp.arange(K * T, dtype=jnp.int32) // K
    xg = _scatter_x(x_flat, slots_flat, tok_idx)

    y = pl.pallas_call(
        _gmm_kernel,
        grid_spec=pltpu.PrefetchScalarGridSpec(
            num_scalar_prefetch=2,
            grid=(NBLK,),
            in_specs=[
                pl.BlockSpec((BM, D), lambda i, be, nb: (jnp.minimum(i, nb[0] - 1), 0)),
                pl.BlockSpec((1, D, 2 * I), lambda i, be, nb: (be[i], 0, 0)),
                pl.BlockSpec((1, I, D), lambda i, be, nb: (be[i], 0, 0)),
            ],
            out_specs=pl.BlockSpec((BM, D),
                                   lambda i, be, nb: (jnp.minimum(i, nb[0] - 1), 0)),
        ),
        out_shape=jax.ShapeDtypeStruct((NPAD, D), jnp.float32),
    )(be_flat, nb_flat, xg, w_gate_up, w_down)

    y_gath = _gather_y(y, slots_flat)

    routed = pl.pallas_call(
        _combine_kernel,
        grid=(4,),
        in_specs=[
            pl.BlockSpec((T // 4, K, D), lambda t: (t, 0, 0)),
            pl.BlockSpec((T // 4, K), lambda t: (t, 0)),
        ],
        out_specs=pl.BlockSpec((T // 4, D), lambda t: (t, 0)),
        out_shape=jax.ShapeDtypeStruct((T, D), jnp.float32),
    )(y_gath.reshape(T, K, D), wts2)

    return routed.reshape(x.shape), counts.reshape(E), ent.reshape(())
